# baseline (speedup 1.0000x reference)
import jax
import jax.numpy as jnp
from jax import lax
from jax.experimental import pallas as pl
from jax.experimental.pallas import tpu as pltpu
from jax.experimental.pallas import tpu_sc as plsc

N = 32768
NSEG = 16
L = 16
NW = 32
C = N // NW
VPC = C // L
W0 = 1024
W1 = C + 128


def _pw_body(val_hbm, off_hbm, pw_hbm, valout_hbm, out_hbm,
             off_v, win_v, out_v, val_v, sem0, sem1):
    wid = lax.axis_index("s") * 2 + lax.axis_index("c")
    base = wid * C
    c_off = pltpu.async_copy(off_hbm.at[pl.ds(0, L)], off_v, sem0)
    c_w0 = pltpu.async_copy(pw_hbm.at[pl.ds(0, W0)],
                            win_v.at[pl.ds(0, W0)], sem1)
    c_val = pltpu.async_copy(val_hbm.at[pl.ds(base, C)], val_v, sem1)
    c_off.wait()
    ov = off_v[...]
    s0 = jnp.int32(0)
    bvecs = []
    for b in range(NSEG):
        ob = ov[b]
        s0 = jnp.where(ob <= base, ob, s0)
        bvecs.append(lax.broadcast(ob, (L,)))
    a0 = base - s0
    a0_al = pl.multiple_of(
        jnp.minimum(jnp.bitwise_and(a0, -128), N - W1), 128)
    c_w1 = pltpu.async_copy(pw_hbm.at[pl.ds(a0_al, W1)],
                            win_v.at[pl.ds(W0, W1)], sem0)
    c_w0.wait()
    c_w1.wait()
    lane = lax.broadcasted_iota(jnp.int32, (L,), 0)
    shift = W0 - a0_al
    for v in range(VPC):
        pos = lane + (base + v * L)
        diffs = [plsc.bitcast(pos - bv, jnp.uint32) for bv in bvecs]
        while len(diffs) > 1:
            diffs = [jnp.minimum(diffs[i], diffs[i + 1])
                     for i in range(0, len(diffs), 2)]
        seq = plsc.bitcast(diffs[0], jnp.int32)
        idx = jnp.where(seq < W0, seq, seq + shift)
        out_v[pl.ds(v * L, L)] = plsc.load_gather(win_v, [idx])
    a_out = pltpu.async_copy(out_v, out_hbm.at[pl.ds(base, C)], sem0)
    c_val.wait()
    a_val = pltpu.async_copy(val_v, valout_hbm.at[pl.ds(base, C)], sem1)
    a_out.wait()
    a_val.wait()


@jax.jit
def _position_weights(values, offsets, position_weight):
    mesh = plsc.VectorSubcoreMesh(core_axis_name="c", subcore_axis_name="s")
    f = pl.kernel(
        _pw_body,
        out_type=(jax.ShapeDtypeStruct((N,), jnp.int32),
                  jax.ShapeDtypeStruct((N,), jnp.float32)),
        mesh=mesh,
        scratch_types=[
            pltpu.VMEM((L,), jnp.int32),
            pltpu.VMEM((W0 + W1,), jnp.float32),
            pltpu.VMEM((C,), jnp.float32),
            pltpu.VMEM((C,), jnp.int32),
            pltpu.SemaphoreType.DMA,
            pltpu.SemaphoreType.DMA,
        ],
        compiler_params=pltpu.CompilerParams(needs_layout_passes=False),
    )
    return f(values, offsets, position_weight)


def kernel(values, offsets, position_weight):
    values_out, weights = _position_weights(values, offsets, position_weight)
    return values_out, weights
